# baseline (speedup 1.0000x reference)
import functools

import jax
import jax.numpy as jnp
from jax import lax
from jax.experimental import pallas as pl
from jax.experimental.pallas import tpu as pltpu
from jax.experimental.pallas import tpu_sc as plsc

NW = 32
LANES = 16
CHUNK = 8192


def _sc_hist_kernel(llen, n):
    per_w = n // NW
    n_chunks = per_w // CHUNK
    half = llen // 2
    mesh = plsc.VectorSubcoreMesh(core_axis_name="c", subcore_axis_name="s")

    @functools.partial(
        pl.kernel,
        out_type=jax.ShapeDtypeStruct((NW, half), jnp.int32),
        mesh=mesh,
        compiler_params=pltpu.CompilerParams(needs_layout_passes=False),
        scratch_types=[
            pltpu.VMEM((llen,), jnp.int32),
            pltpu.VMEM((half,), jnp.int32),
            pltpu.VMEM((CHUNK,), jnp.int32),
            pltpu.VMEM((CHUNK,), jnp.int32),
            pltpu.SemaphoreType.DMA,
            pltpu.SemaphoreType.DMA,
            pltpu.SemaphoreType.DMA,
        ],
    )
    def sc_hist(label_hbm, gd_hbm, out_hbm, gd_v, hist_v, lab0_v, lab1_v,
                gsem, sem0, sem1):
        wid = lax.axis_index("c") * 16 + lax.axis_index("s")
        bufs = (lab0_v, lab1_v)
        sems = (sem0, sem1)

        def start(c):
            base = wid * per_w + c * CHUNK
            return pltpu.async_copy(
                label_hbm.at[pl.ds(base, CHUNK)], bufs[c % 2], sems[c % 2])

        gcopy = pltpu.async_copy(gd_hbm, gd_v, gsem)
        handles = {0: start(0)}
        if n_chunks > 1:
            handles[1] = start(1)

        zero = jnp.zeros((LANES,), jnp.int32)

        @plsc.parallel_loop(0, half, LANES, unroll=16)
        def zbody(i):
            hist_v[pl.ds(i, LANES)] = zero

        gcopy.wait()
        one = jnp.full((LANES,), 1, jnp.int32)

        for c in range(n_chunks):
            handles[c].wait()
            lab_v = bufs[c % 2]

            @plsc.parallel_loop(0, CHUNK, LANES, unroll=16)
            def gbody(i):
                lab = lab_v[pl.ds(i, LANES)]
                mapped = plsc.load_gather(gd_v, [lab])
                word = mapped & 0x7FFF
                inc = lax.shift_left(
                    one, lax.shift_left(lax.shift_right_logical(mapped, 15), 4))
                plsc.addupdate_scatter(hist_v, [word], inc)

            if c + 2 < n_chunks:
                handles[c + 2] = start(c + 2)

        pltpu.sync_copy(hist_v, out_hbm.at[wid])

    return sc_hist


def _tc_reduce_kernel(llen, n_f):
    BLK = 8192
    half = llen // 2
    nlo = half // BLK

    def body(total_ref, part_ref, cnts_ref, out_ref):
        i = pl.program_id(0)
        tot = total_ref[0, 0] + n_f
        p = part_ref[...]
        h = jnp.where(i < nlo, p & 0xFFFF, lax.shift_right_logical(p, 16))
        s = jnp.sum(h, axis=0).astype(jnp.float32)
        out_ref[...] = jnp.maximum((s + cnts_ref[...]) / tot, 0.01)

    return pl.pallas_call(
        body,
        grid=(2 * nlo,),
        in_specs=[
            pl.BlockSpec(memory_space=pltpu.SMEM),
            pl.BlockSpec((NW, BLK), lambda i: (0, i % nlo)),
            pl.BlockSpec((BLK,), lambda i: (i,)),
        ],
        out_specs=pl.BlockSpec((BLK,), lambda i: (i,)),
        out_shape=jax.ShapeDtypeStruct((llen,), jnp.float32),
    )


def kernel(gdict, flatten_label, llen, cnts, total):
    llen_static = gdict.shape[0]
    n = flatten_label.shape[0]

    partials = _sc_hist_kernel(llen_static, n)(
        flatten_label, gdict.astype(jnp.int32))

    total2d = jnp.reshape(total.astype(jnp.float32), (1, 1))
    return _tc_reduce_kernel(llen_static, float(n))(
        total2d, partials, cnts[:llen_static])


# speedup vs baseline: 1.0013x; 1.0013x over previous
import functools

import jax
import jax.numpy as jnp
from jax import lax
from jax.experimental import pallas as pl
from jax.experimental.pallas import tpu as pltpu
from jax.experimental.pallas import tpu_sc as plsc

NW = 32
LANES = 16
CHUNK = 8192


def _sc_hist_kernel(llen, n):
    per_w = n // NW
    n_chunks = per_w // CHUNK
    half = llen // 2
    mesh = plsc.VectorSubcoreMesh(core_axis_name="c", subcore_axis_name="s")

    @functools.partial(
        pl.kernel,
        out_type=jax.ShapeDtypeStruct((NW, half), jnp.int32),
        mesh=mesh,
        compiler_params=pltpu.CompilerParams(needs_layout_passes=False),
        scratch_types=[
            pltpu.VMEM((llen,), jnp.int32),
            pltpu.VMEM((half,), jnp.int32),
            pltpu.VMEM((CHUNK,), jnp.int32),
            pltpu.VMEM((CHUNK,), jnp.int32),
            pltpu.SemaphoreType.DMA,
            pltpu.SemaphoreType.DMA,
            pltpu.SemaphoreType.DMA,
        ],
    )
    def sc_hist(label_hbm, gd_hbm, out_hbm, gd_v, hist_v, lab0_v, lab1_v,
                gsem, sem0, sem1):
        wid = lax.axis_index("c") * 16 + lax.axis_index("s")
        bufs = (lab0_v, lab1_v)
        sems = (sem0, sem1)

        def start(c):
            base = wid * per_w + c * CHUNK
            return pltpu.async_copy(
                label_hbm.at[pl.ds(base, CHUNK)], bufs[c % 2], sems[c % 2])

        gcopy = pltpu.async_copy(gd_hbm, gd_v, gsem)
        handles = {0: start(0)}
        if n_chunks > 1:
            handles[1] = start(1)

        zero = jnp.zeros((LANES,), jnp.int32)

        @plsc.parallel_loop(0, half, LANES, unroll=16)
        def zbody(i):
            hist_v[pl.ds(i, LANES)] = zero

        gcopy.wait()
        one = jnp.full((LANES,), 1, jnp.int32)

        for c in range(n_chunks):
            handles[c].wait()
            lab_v = bufs[c % 2]

            @plsc.parallel_loop(0, CHUNK, LANES, unroll=8)
            def gbody(i):
                lab = lab_v[pl.ds(i, LANES)]
                mapped = plsc.load_gather(gd_v, [lab])
                word = mapped & 0x7FFF
                inc = lax.shift_left(
                    one, lax.shift_left(lax.shift_right_logical(mapped, 15), 4))
                plsc.addupdate_scatter(hist_v, [word], inc)

            if c + 2 < n_chunks:
                handles[c + 2] = start(c + 2)

        pltpu.sync_copy(hist_v, out_hbm.at[wid])

    return sc_hist


def _tc_reduce_kernel(llen, n_f):
    BLK = 8192
    half = llen // 2
    nlo = half // BLK

    def body(total_ref, part_ref, cnts_ref, out_ref):
        i = pl.program_id(0)
        tot = total_ref[0, 0] + n_f
        p = part_ref[...]
        h = jnp.where(i < nlo, p & 0xFFFF, lax.shift_right_logical(p, 16))
        s = jnp.sum(h, axis=0).astype(jnp.float32)
        out_ref[...] = jnp.maximum((s + cnts_ref[...]) / tot, 0.01)

    return pl.pallas_call(
        body,
        grid=(2 * nlo,),
        in_specs=[
            pl.BlockSpec(memory_space=pltpu.SMEM),
            pl.BlockSpec((NW, BLK), lambda i: (0, i % nlo)),
            pl.BlockSpec((BLK,), lambda i: (i,)),
        ],
        out_specs=pl.BlockSpec((BLK,), lambda i: (i,)),
        out_shape=jax.ShapeDtypeStruct((llen,), jnp.float32),
    )


def kernel(gdict, flatten_label, llen, cnts, total):
    llen_static = gdict.shape[0]
    n = flatten_label.shape[0]

    partials = _sc_hist_kernel(llen_static, n)(
        flatten_label, gdict.astype(jnp.int32))

    total2d = jnp.reshape(total.astype(jnp.float32), (1, 1))
    return _tc_reduce_kernel(llen_static, float(n))(
        total2d, partials, cnts[:llen_static])


# speedup vs baseline: 1.0354x; 1.0340x over previous
import functools

import jax
import jax.numpy as jnp
from jax import lax
from jax.experimental import pallas as pl
from jax.experimental.pallas import tpu as pltpu
from jax.experimental.pallas import tpu_sc as plsc

NW = 32
LANES = 16
CHUNK = 8192


def _sc_hist_kernel(llen, n):
    per_w = n // NW
    n_chunks = per_w // CHUNK
    half = llen // 2
    mesh = plsc.VectorSubcoreMesh(core_axis_name="c", subcore_axis_name="s")

    @functools.partial(
        pl.kernel,
        out_type=jax.ShapeDtypeStruct((NW, half), jnp.int32),
        mesh=mesh,
        compiler_params=pltpu.CompilerParams(needs_layout_passes=False),
        scratch_types=[
            pltpu.VMEM((llen,), jnp.int32),
            pltpu.VMEM((half,), jnp.int32),
            pltpu.VMEM((CHUNK,), jnp.int32),
            pltpu.VMEM((CHUNK,), jnp.int32),
            pltpu.SemaphoreType.DMA,
            pltpu.SemaphoreType.DMA,
            pltpu.SemaphoreType.DMA,
        ],
    )
    def sc_hist(label_hbm, gd_hbm, out_hbm, gd_v, hist_v, lab0_v, lab1_v,
                gsem, sem0, sem1):
        wid = lax.axis_index("c") * 16 + lax.axis_index("s")
        bufs = (lab0_v, lab1_v)
        sems = (sem0, sem1)

        def start(c):
            base = wid * per_w + c * CHUNK
            return pltpu.async_copy(
                label_hbm.at[pl.ds(base, CHUNK)], bufs[c % 2], sems[c % 2])

        gcopy = pltpu.async_copy(gd_hbm, gd_v, gsem)
        handles = {0: start(0)}
        if n_chunks > 1:
            handles[1] = start(1)

        zero = jnp.zeros((LANES,), jnp.int32)

        @plsc.parallel_loop(0, half, LANES, unroll=16)
        def zbody(i):
            hist_v[pl.ds(i, LANES)] = zero

        gcopy.wait()
        one = jnp.full((LANES,), 1, jnp.int32)
        hi_one = jnp.full((LANES,), 1 << 16, jnp.int32)

        for c in range(n_chunks):
            handles[c].wait()
            lab_v = bufs[c % 2]

            @plsc.parallel_loop(0, CHUNK, LANES, unroll=8)
            def gbody(i):
                lab = lab_v[pl.ds(i, LANES)]
                mapped = plsc.load_gather(gd_v, [lab])
                word = mapped & 0x7FFF
                inc = jnp.where(mapped < half, one, hi_one)
                plsc.addupdate_scatter(hist_v, [word], inc)

            if c + 2 < n_chunks:
                handles[c + 2] = start(c + 2)

        pltpu.sync_copy(hist_v, out_hbm.at[wid])

    return sc_hist


def _tc_reduce_kernel(llen, n_f):
    BLK = 8192
    half = llen // 2
    grid = half // BLK

    def body(total_ref, part_ref, cnts_ref, out_ref):
        tot = total_ref[0, 0] + n_f
        p = part_ref[...]
        s_lo = jnp.sum(p & 0xFFFF, axis=0).astype(jnp.float32)
        s_hi = jnp.sum(lax.shift_right_logical(p, 16), axis=0).astype(jnp.float32)
        out_ref[0, :] = jnp.maximum((s_lo + cnts_ref[0, :]) / tot, 0.01)
        out_ref[1, :] = jnp.maximum((s_hi + cnts_ref[1, :]) / tot, 0.01)

    return pl.pallas_call(
        body,
        grid=(grid,),
        in_specs=[
            pl.BlockSpec(memory_space=pltpu.SMEM),
            pl.BlockSpec((NW, BLK), lambda i: (0, i)),
            pl.BlockSpec((2, BLK), lambda i: (0, i)),
        ],
        out_specs=pl.BlockSpec((2, BLK), lambda i: (0, i)),
        out_shape=jax.ShapeDtypeStruct((2, half), jnp.float32),
    )


def kernel(gdict, flatten_label, llen, cnts, total):
    llen_static = gdict.shape[0]
    n = flatten_label.shape[0]

    partials = _sc_hist_kernel(llen_static, n)(
        flatten_label, gdict.astype(jnp.int32))

    total2d = jnp.reshape(total.astype(jnp.float32), (1, 1))
    cnts2d = jnp.reshape(cnts[:llen_static], (2, llen_static // 2))
    out = _tc_reduce_kernel(llen_static, float(n))(total2d, partials, cnts2d)
    return jnp.reshape(out, (llen_static,))


# speedup vs baseline: 1.0563x; 1.0202x over previous
import functools

import jax
import jax.numpy as jnp
from jax import lax
from jax.experimental import pallas as pl
from jax.experimental.pallas import tpu as pltpu
from jax.experimental.pallas import tpu_sc as plsc

NW = 32
LANES = 16
CHUNK = 8192


def _sc_hist_kernel(llen, n):
    per_w = n // NW
    n_chunks = per_w // CHUNK
    half = llen // 2
    mesh = plsc.VectorSubcoreMesh(core_axis_name="c", subcore_axis_name="s")

    @functools.partial(
        pl.kernel,
        out_type=jax.ShapeDtypeStruct((NW, half), jnp.int32),
        mesh=mesh,
        compiler_params=pltpu.CompilerParams(needs_layout_passes=False),
        scratch_types=[
            pltpu.VMEM((llen,), jnp.int32),
            pltpu.VMEM((half,), jnp.int32),
            pltpu.VMEM((CHUNK,), jnp.int32),
            pltpu.VMEM((CHUNK,), jnp.int32),
            pltpu.SemaphoreType.DMA,
            pltpu.SemaphoreType.DMA,
            pltpu.SemaphoreType.DMA,
        ],
    )
    def sc_hist(label_hbm, gd_hbm, out_hbm, gd_v, hist_v, lab0_v, lab1_v,
                gsem, sem0, sem1):
        wid = lax.axis_index("c") * 16 + lax.axis_index("s")
        bufs = (lab0_v, lab1_v)
        sems = (sem0, sem1)

        def start(c):
            base = wid * per_w + c * CHUNK
            return pltpu.async_copy(
                label_hbm.at[pl.ds(base, CHUNK)], bufs[c % 2], sems[c % 2])

        gcopy = pltpu.async_copy(gd_hbm, gd_v, gsem)
        handles = {0: start(0)}
        if n_chunks > 1:
            handles[1] = start(1)

        zero = jnp.zeros((LANES,), jnp.int32)

        @plsc.parallel_loop(0, half, LANES, unroll=16)
        def zbody(i):
            hist_v[pl.ds(i, LANES)] = zero

        gcopy.wait()
        one = jnp.full((LANES,), 1, jnp.int32)
        hi_one = jnp.full((LANES,), 1 << 16, jnp.int32)

        for c in range(n_chunks):
            handles[c].wait()
            lab_v = bufs[c % 2]

            @plsc.parallel_loop(0, CHUNK, LANES, unroll=8)
            def gbody(i):
                lab = lab_v[pl.ds(i, LANES)]
                mapped = plsc.load_gather(gd_v, [lab])
                word = mapped & 0x7FFF
                inc = jnp.where(mapped < half, one, hi_one)
                plsc.addupdate_scatter(hist_v, [word], inc)

            if c + 2 < n_chunks:
                handles[c + 2] = start(c + 2)

        pltpu.sync_copy(hist_v, out_hbm.at[wid])

    return sc_hist


def _tc_reduce_kernel(llen, n_f):
    BLK = 16384
    half = llen // 2
    grid = half // BLK

    def body(total_ref, part_ref, cnts_ref, out_ref):
        tot = total_ref[0, 0] + n_f
        p = part_ref[...]
        s_lo = jnp.sum(p & 0xFFFF, axis=0).astype(jnp.float32)
        s_hi = jnp.sum(lax.shift_right_logical(p, 16), axis=0).astype(jnp.float32)
        out_ref[0, :] = jnp.maximum((s_lo + cnts_ref[0, :]) / tot, 0.01)
        out_ref[1, :] = jnp.maximum((s_hi + cnts_ref[1, :]) / tot, 0.01)

    return pl.pallas_call(
        body,
        grid=(grid,),
        in_specs=[
            pl.BlockSpec(memory_space=pltpu.SMEM),
            pl.BlockSpec((NW, BLK), lambda i: (0, i)),
            pl.BlockSpec((2, BLK), lambda i: (0, i)),
        ],
        out_specs=pl.BlockSpec((2, BLK), lambda i: (0, i)),
        out_shape=jax.ShapeDtypeStruct((2, half), jnp.float32),
    )


def kernel(gdict, flatten_label, llen, cnts, total):
    llen_static = gdict.shape[0]
    n = flatten_label.shape[0]

    partials = _sc_hist_kernel(llen_static, n)(
        flatten_label, gdict.astype(jnp.int32))

    total2d = jnp.reshape(total.astype(jnp.float32), (1, 1))
    cnts2d = jnp.reshape(cnts[:llen_static], (2, llen_static // 2))
    out = _tc_reduce_kernel(llen_static, float(n))(total2d, partials, cnts2d)
    return jnp.reshape(out, (llen_static,))


# speedup vs baseline: 1.0613x; 1.0047x over previous
import functools

import jax
import jax.numpy as jnp
from jax import lax
from jax.experimental import pallas as pl
from jax.experimental.pallas import tpu as pltpu
from jax.experimental.pallas import tpu_sc as plsc

NW = 32
LANES = 16
CHUNK = 8192


def _sc_hist_kernel(llen, n):
    per_w = n // NW
    n_chunks = per_w // CHUNK
    half = llen // 2
    mesh = plsc.VectorSubcoreMesh(core_axis_name="c", subcore_axis_name="s")

    @functools.partial(
        pl.kernel,
        out_type=jax.ShapeDtypeStruct((NW, half), jnp.int32),
        mesh=mesh,
        compiler_params=pltpu.CompilerParams(needs_layout_passes=False),
        scratch_types=[
            pltpu.VMEM((llen,), jnp.int32),
            pltpu.VMEM((half,), jnp.int32),
            pltpu.VMEM((CHUNK,), jnp.int32),
            pltpu.VMEM((CHUNK,), jnp.int32),
            pltpu.SemaphoreType.DMA,
            pltpu.SemaphoreType.DMA,
            pltpu.SemaphoreType.DMA,
        ],
    )
    def sc_hist(label_hbm, gd_hbm, out_hbm, gd_v, hist_v, lab0_v, lab1_v,
                gsem, sem0, sem1):
        wid = lax.axis_index("c") * 16 + lax.axis_index("s")
        bufs = (lab0_v, lab1_v)
        sems = (sem0, sem1)

        def start(c):
            base = wid * per_w + c * CHUNK
            return pltpu.async_copy(
                label_hbm.at[pl.ds(base, CHUNK)], bufs[c % 2], sems[c % 2])

        gcopy = pltpu.async_copy(gd_hbm, gd_v, gsem)
        handles = {0: start(0)}
        if n_chunks > 1:
            handles[1] = start(1)

        zero = jnp.zeros((LANES,), jnp.int32)

        @plsc.parallel_loop(0, half, LANES, unroll=16)
        def zbody(i):
            hist_v[pl.ds(i, LANES)] = zero

        gcopy.wait()
        one = jnp.full((LANES,), 1, jnp.int32)
        hi_one = jnp.full((LANES,), 1 << 16, jnp.int32)

        for c in range(n_chunks):
            handles[c].wait()
            lab_v = bufs[c % 2]

            @plsc.parallel_loop(0, CHUNK, LANES, unroll=8)
            def gbody(i):
                lab = lab_v[pl.ds(i, LANES)]
                mapped = plsc.load_gather(gd_v, [lab])
                word = mapped & 0x7FFF
                inc = jnp.where(mapped < half, one, hi_one)
                plsc.addupdate_scatter(hist_v, [word], inc)

            if c + 2 < n_chunks:
                handles[c + 2] = start(c + 2)

        pltpu.sync_copy(hist_v, out_hbm.at[wid])

    return sc_hist


def _tc_reduce_kernel(llen, n_f):
    BLK = 16384
    half = llen // 2
    grid = half // BLK

    def body(total_ref, part_ref, cnts_lo_ref, cnts_hi_ref, out_ref):
        tot = total_ref[0, 0] + n_f
        p = part_ref[...]
        s_lo = jnp.sum(p & 0xFFFF, axis=0).astype(jnp.float32)
        s_hi = jnp.sum(lax.shift_right_logical(p, 16), axis=0).astype(jnp.float32)
        out_ref[0, :] = jnp.maximum((s_lo + cnts_lo_ref[...]) / tot, 0.01)
        out_ref[1, :] = jnp.maximum((s_hi + cnts_hi_ref[...]) / tot, 0.01)

    return pl.pallas_call(
        body,
        grid=(grid,),
        in_specs=[
            pl.BlockSpec(memory_space=pltpu.SMEM),
            pl.BlockSpec((NW, BLK), lambda i: (0, i)),
            pl.BlockSpec((BLK,), lambda i: (i,)),
            pl.BlockSpec((BLK,), lambda i: (i + grid,)),
        ],
        out_specs=pl.BlockSpec((2, BLK), lambda i: (0, i)),
        out_shape=jax.ShapeDtypeStruct((2, half), jnp.float32),
    )


def kernel(gdict, flatten_label, llen, cnts, total):
    llen_static = gdict.shape[0]
    n = flatten_label.shape[0]

    partials = _sc_hist_kernel(llen_static, n)(
        flatten_label, gdict.astype(jnp.int32))

    total2d = jnp.reshape(total.astype(jnp.float32), (1, 1))
    out = _tc_reduce_kernel(llen_static, float(n))(total2d, partials, cnts, cnts)
    return jnp.reshape(out, (llen_static,))
